# baseline (speedup 1.0000x reference)
import functools

import jax
import jax.numpy as jnp
from jax import lax
from jax.experimental import pallas as pl
from jax.experimental.pallas import tpu as pltpu
from jax.experimental.pallas import tpu_sc as plsc

_B, _T, _C = 16, 2048, 96
_BLANK = _C - 1
_TBLK = 128
_L = 16
_NC, _NS = 2, 16


def _argmax_body(xt_ref, out_ref):
    for b in range(_B):
        x = xt_ref[b]
        m = jnp.max(x, axis=0, keepdims=True)
        cls = lax.broadcasted_iota(jnp.int32, x.shape, 0)
        out_ref[b, :] = jnp.min(jnp.where(x == m, cls, _C), axis=0)


def _tc_argmax(inputs):
    xt = jnp.transpose(inputs, (0, 2, 1))
    return pl.pallas_call(
        _argmax_body,
        grid=(_T // _TBLK,),
        in_specs=[pl.BlockSpec((_B, _C, _TBLK), lambda t: (0, 0, t))],
        out_specs=pl.BlockSpec((_B, _TBLK), lambda t: (0, t)),
        out_shape=jax.ShapeDtypeStruct((_B, _T), jnp.int32),
    )(xt)


_sc_mesh = plsc.VectorSubcoreMesh(
    core_axis_name="c", subcore_axis_name="s", num_cores=1
)


@functools.partial(
    pl.kernel,
    mesh=_sc_mesh,
    out_type=jax.ShapeDtypeStruct((_B, _T), jnp.int32),
    scratch_types=[
        pltpu.VMEM((_T,), jnp.int32),
        pltpu.VMEM((_T,), jnp.int32),
    ],
    compiler_params=pltpu.CompilerParams(
        needs_layout_passes=False,
        skip_device_barrier=True,
        disable_bounds_checks=True,
        disable_semaphore_checks=True,
    ),
)
def _sc_decode(preds_hbm, out_hbm, row_v, out_v):
    wid = lax.axis_index("s")

    @pl.when(wid < _B)
    def _():
        iota = lax.iota(jnp.int32, _L)
        blank_v = jnp.full((_L,), _BLANK, jnp.int32)

        pltpu.sync_copy(preds_hbm.at[wid], row_v)

        def step(i, carry):
            base = i * _L
            idx = base + iota
            v = row_v[pl.ds(base, _L)]
            p = plsc.load_gather(row_v, [jnp.maximum(idx - 1, 0)])
            keep = ((v != p) | (idx == 0)) & (v != _BLANK)
            ks = plsc.cumsum(keep.astype(jnp.int32))
            pos = jnp.maximum(carry + ks, 0)
            plsc.store_scatter(out_v, [pos], v, mask=keep)
            return carry + plsc.all_reduce_population_count(keep)

        carry = lax.fori_loop(
            0, _T // _L, step, jnp.full((_L,), -1, jnp.int32), unroll=4
        )

        cnt_vec = carry + 1

        def fill(j, c):
            idx = j * _L + iota
            plsc.store_scatter(out_v, [idx], blank_v, mask=idx >= cnt_vec)
            return c

        lax.fori_loop(0, _T // _L, fill, 0, unroll=4)
        pltpu.sync_copy(out_v, out_hbm.at[wid])


def kernel(inputs):
    preds = _tc_argmax(inputs)
    out = _sc_decode(preds)
    return out.astype(jnp.int64)


# speedup vs baseline: 1.2142x; 1.2142x over previous
import functools

import jax
import jax.numpy as jnp
from jax import lax
from jax.experimental import pallas as pl
from jax.experimental.pallas import tpu as pltpu
from jax.experimental.pallas import tpu_sc as plsc

_B, _T, _C = 16, 2048, 96
_BLANK = _C - 1
_TBLK = 512
_L = 16
_NC, _NS = 2, 16


def _argmax_body(xt_ref, out_ref):
    for b in range(_B):
        x = xt_ref[b]
        m = jnp.max(x, axis=0, keepdims=True)
        cls = lax.broadcasted_iota(jnp.int32, x.shape, 0)
        out_ref[b, :] = jnp.min(jnp.where(x == m, cls, _C), axis=0)


def _tc_argmax(inputs):
    xt = jnp.transpose(inputs, (0, 2, 1))
    return pl.pallas_call(
        _argmax_body,
        grid=(_T // _TBLK,),
        in_specs=[pl.BlockSpec((_B, _C, _TBLK), lambda t: (0, 0, t))],
        out_specs=pl.BlockSpec((_B, _TBLK), lambda t: (0, t)),
        out_shape=jax.ShapeDtypeStruct((_B, _T), jnp.int32),
    )(xt)


_sc_mesh = plsc.VectorSubcoreMesh(
    core_axis_name="c", subcore_axis_name="s", num_cores=1
)


@functools.partial(
    pl.kernel,
    mesh=_sc_mesh,
    out_type=jax.ShapeDtypeStruct((_B, _T), jnp.int32),
    scratch_types=[
        pltpu.VMEM((_T,), jnp.int32),
        pltpu.VMEM((_T,), jnp.int32),
    ],
    compiler_params=pltpu.CompilerParams(
        needs_layout_passes=False,
        skip_device_barrier=True,
        disable_bounds_checks=True,
        disable_semaphore_checks=True,
    ),
)
def _sc_decode(preds_hbm, out_hbm, row_v, out_v):
    wid = lax.axis_index("s")

    @pl.when(wid < _B)
    def _():
        iota = lax.iota(jnp.int32, _L)
        blank_v = jnp.full((_L,), _BLANK, jnp.int32)

        pltpu.sync_copy(preds_hbm.at[wid], row_v)

        def step(i, carry):
            base = i * _L
            idx = base + iota
            v = row_v[pl.ds(base, _L)]
            p = plsc.load_gather(row_v, [jnp.maximum(idx - 1, 0)])
            keep = ((v != p) | (idx == 0)) & (v != _BLANK)
            ks = plsc.cumsum(keep.astype(jnp.int32))
            pos = jnp.maximum(carry + ks, 0)
            plsc.store_scatter(out_v, [pos], v, mask=keep)
            return carry + plsc.all_reduce_population_count(keep)

        carry = lax.fori_loop(
            0, _T // _L, step, jnp.full((_L,), -1, jnp.int32), unroll=4
        )

        cnt_vec = carry + 1

        def fill(j, c):
            idx = j * _L + iota
            plsc.store_scatter(out_v, [idx], blank_v, mask=idx >= cnt_vec)
            return c

        lax.fori_loop(0, _T // _L, fill, 0, unroll=4)
        pltpu.sync_copy(out_v, out_hbm.at[wid])


def kernel(inputs):
    preds = _tc_argmax(inputs)
    out = _sc_decode(preds)
    return out.astype(jnp.int64)


# speedup vs baseline: 1.2308x; 1.0137x over previous
import functools

import jax
import jax.numpy as jnp
from jax import lax
from jax.experimental import pallas as pl
from jax.experimental.pallas import tpu as pltpu
from jax.experimental.pallas import tpu_sc as plsc

_B, _T, _C = 16, 2048, 96
_BLANK = _C - 1
_TBLK = 1024
_L = 16
_NC, _NS = 2, 16


def _argmax_body(xt_ref, out_ref):
    for b in range(_B):
        x = xt_ref[b]
        m = jnp.max(x, axis=0, keepdims=True)
        cls = lax.broadcasted_iota(jnp.int32, x.shape, 0)
        out_ref[b, :] = jnp.min(jnp.where(x == m, cls, _C), axis=0)


def _tc_argmax(inputs):
    xt = jnp.transpose(inputs, (0, 2, 1))
    return pl.pallas_call(
        _argmax_body,
        grid=(_T // _TBLK,),
        in_specs=[pl.BlockSpec((_B, _C, _TBLK), lambda t: (0, 0, t))],
        out_specs=pl.BlockSpec((_B, _TBLK), lambda t: (0, t)),
        out_shape=jax.ShapeDtypeStruct((_B, _T), jnp.int32),
    )(xt)


_sc_mesh = plsc.VectorSubcoreMesh(
    core_axis_name="c", subcore_axis_name="s", num_cores=1
)


@functools.partial(
    pl.kernel,
    mesh=_sc_mesh,
    out_type=jax.ShapeDtypeStruct((_B, _T), jnp.int32),
    scratch_types=[
        pltpu.VMEM((_T,), jnp.int32),
        pltpu.VMEM((_T,), jnp.int32),
    ],
    compiler_params=pltpu.CompilerParams(
        needs_layout_passes=False,
        skip_device_barrier=True,
        disable_bounds_checks=True,
        disable_semaphore_checks=True,
    ),
)
def _sc_decode(preds_hbm, out_hbm, row_v, out_v):
    wid = lax.axis_index("s")

    @pl.when(wid < _B)
    def _():
        iota = lax.iota(jnp.int32, _L)
        blank_v = jnp.full((_L,), _BLANK, jnp.int32)

        pltpu.sync_copy(preds_hbm.at[wid], row_v)

        def step(i, carry):
            base = i * _L
            idx = base + iota
            v = row_v[pl.ds(base, _L)]
            p = plsc.load_gather(row_v, [jnp.maximum(idx - 1, 0)])
            keep = ((v != p) | (idx == 0)) & (v != _BLANK)
            ks = plsc.cumsum(keep.astype(jnp.int32))
            pos = jnp.maximum(carry + ks, 0)
            plsc.store_scatter(out_v, [pos], v, mask=keep)
            return carry + plsc.all_reduce_population_count(keep)

        carry = lax.fori_loop(
            0, _T // _L, step, jnp.full((_L,), -1, jnp.int32), unroll=4
        )

        cnt_vec = carry + 1

        def fill(j, c):
            idx = j * _L + iota
            plsc.store_scatter(out_v, [idx], blank_v, mask=idx >= cnt_vec)
            return c

        lax.fori_loop(0, _T // _L, fill, 0, unroll=4)
        pltpu.sync_copy(out_v, out_hbm.at[wid])


def kernel(inputs):
    preds = _tc_argmax(inputs)
    out = _sc_decode(preds)
    return out.astype(jnp.int64)


# speedup vs baseline: 1.2346x; 1.0031x over previous
import functools

import jax
import jax.numpy as jnp
from jax import lax
from jax.experimental import pallas as pl
from jax.experimental.pallas import tpu as pltpu
from jax.experimental.pallas import tpu_sc as plsc

_B, _T, _C = 16, 2048, 96
_BLANK = _C - 1
_TBLK = 1024
_L = 16
_NC, _NS = 2, 16


def _argmax_body(xt_ref, out_ref):
    for b in range(_B):
        x = xt_ref[b]
        m = jnp.max(x, axis=0, keepdims=True)
        cls = lax.broadcasted_iota(jnp.int32, x.shape, 0)
        out_ref[b, :] = jnp.min(jnp.where(x == m, cls, _C), axis=0)


def _tc_argmax(inputs):
    xt = jnp.transpose(inputs, (0, 2, 1))
    return pl.pallas_call(
        _argmax_body,
        grid=(_T // _TBLK,),
        in_specs=[pl.BlockSpec((_B, _C, _TBLK), lambda t: (0, 0, t))],
        out_specs=pl.BlockSpec((_B, _TBLK), lambda t: (0, t)),
        out_shape=jax.ShapeDtypeStruct((_B, _T), jnp.int32),
    )(xt)


_sc_mesh = plsc.VectorSubcoreMesh(
    core_axis_name="c", subcore_axis_name="s", num_cores=1
)


@functools.partial(
    pl.kernel,
    mesh=_sc_mesh,
    out_type=jax.ShapeDtypeStruct((_B, _T), jnp.int32),
    scratch_types=[
        pltpu.VMEM((_T,), jnp.int32),
        pltpu.VMEM((_T,), jnp.int32),
    ],
    compiler_params=pltpu.CompilerParams(
        needs_layout_passes=False,
        skip_device_barrier=True,
        disable_bounds_checks=True,
        disable_semaphore_checks=True,
    ),
)
def _sc_decode(preds_hbm, out_hbm, row_v, out_v):
    wid = lax.axis_index("s")

    @pl.when(wid < _B)
    def _():
        iota = lax.iota(jnp.int32, _L)
        blank_v = jnp.full((_L,), _BLANK, jnp.int32)

        pltpu.sync_copy(preds_hbm.at[wid], row_v)

        def step(i, carry):
            base = i * _L
            idx = base + iota
            v = row_v[pl.ds(base, _L)]
            p = plsc.load_gather(row_v, [jnp.maximum(idx - 1, 0)])
            keep = ((v != p) | (idx == 0)) & (v != _BLANK)
            ks = plsc.cumsum(keep.astype(jnp.int32))
            pos = jnp.maximum(carry + ks, 0)
            plsc.store_scatter(out_v, [pos], v, mask=keep)
            return carry + plsc.all_reduce_population_count(keep)

        carry = lax.fori_loop(
            0, _T // _L, step, jnp.full((_L,), -1, jnp.int32), unroll=4
        )

        cnt_vec = carry + 1
        cnt = carry[0] + 1

        def fill(j, c):
            idx = j * _L + iota
            plsc.store_scatter(out_v, [idx], blank_v, mask=idx >= cnt_vec)
            return c

        lax.fori_loop(cnt // _L, _T // _L, fill, 0)
        pltpu.sync_copy(out_v, out_hbm.at[wid])


def kernel(inputs):
    preds = _tc_argmax(inputs)
    out = _sc_decode(preds)
    return out.astype(jnp.int64)
